# baseline (speedup 1.0000x reference)
import jax
import jax.numpy as jnp
from jax.experimental import pallas as pl
from jax.experimental.pallas import tpu as pltpu

_N_SPLIT = 8


def _copy_body(x_ref, o_ref, sems):
    rows = x_ref.shape[0]
    chunk = rows // _N_SPLIT
    copies = []
    for i in range(_N_SPLIT):
        lo = i * chunk
        hi = rows if i == _N_SPLIT - 1 else lo + chunk
        copies.append(
            pltpu.make_async_copy(
                x_ref.at[pl.ds(lo, hi - lo)],
                o_ref.at[pl.ds(lo, hi - lo)],
                sems.at[i],
            )
        )
    for c in copies:
        c.start()
    for c in copies:
        c.wait()


def kernel(x):
    orig_shape = x.shape
    flat = x.reshape(576, 32768)
    out = pl.pallas_call(
        _copy_body,
        out_shape=jax.ShapeDtypeStruct(flat.shape, flat.dtype),
        in_specs=[pl.BlockSpec(memory_space=pl.MemorySpace.ANY)],
        out_specs=pl.BlockSpec(memory_space=pl.MemorySpace.ANY),
        scratch_shapes=[pltpu.SemaphoreType.DMA((_N_SPLIT,))],
    )(flat)
    return out.reshape(orig_shape)


# speedup vs baseline: 11.5095x; 11.5095x over previous
import jax
import jax.numpy as jnp
from jax.experimental import pallas as pl
from jax.experimental.pallas import tpu as pltpu

_ROWS = 576
_COLS = 32768
_BLOCK_ROWS = 64


def _copy_body(x_ref, o_ref):
    o_ref[...] = x_ref[...]


def kernel(x):
    orig_shape = x.shape
    flat = x.reshape(_ROWS, _COLS)
    grid = (_ROWS // _BLOCK_ROWS,)
    out = pl.pallas_call(
        _copy_body,
        out_shape=jax.ShapeDtypeStruct(flat.shape, flat.dtype),
        grid=grid,
        in_specs=[pl.BlockSpec((_BLOCK_ROWS, _COLS), lambda i: (i, 0))],
        out_specs=pl.BlockSpec((_BLOCK_ROWS, _COLS), lambda i: (i, 0)),
        compiler_params=pltpu.CompilerParams(
            dimension_semantics=("arbitrary",),
        ),
    )(flat)
    return out.reshape(orig_shape)
